# baseline (speedup 1.0000x reference)
import functools

import jax
import jax.numpy as jnp
from jax import lax
from jax.experimental import pallas as pl
from jax.experimental.pallas import tpu as pltpu
from jax.experimental.pallas import tpu_sc as plsc

VOCAB = 1_000_000
EMBED_DIM = 32
BATCH = 16384
HIST = 50

_NC = 2
_NS = 16
_NW = _NC * _NS

_B = BATCH * HIST
_BW = BATCH // _NW
_C = 256
_NBUF = 2
_NCHUNKS = HIST * (_BW // _C)
_QPH = _BW // _C

_mesh = plsc.VectorSubcoreMesh(core_axis_name="c", subcore_axis_name="s")


@functools.partial(
    pl.kernel,
    mesh=_mesh,
    out_type=jax.ShapeDtypeStruct((_B // 4, 128), jnp.float32),
    scratch_types=[
        pltpu.VMEM((HIST, _BW), jnp.int32),
        [pltpu.VMEM((_C,), jnp.int32) for _ in range(_NBUF)],
        [pltpu.VMEM((_C, 128), jnp.float32) for _ in range(_NBUF)],
        [pltpu.VMEM((_C // 4, 128), jnp.float32) for _ in range(_NBUF)],
        [pltpu.SemaphoreType.DMA for _ in range(_NBUF)],
        [pltpu.SemaphoreType.DMA for _ in range(_NBUF)],
        pltpu.SemaphoreType.DMA,
    ],
    compiler_params=pltpu.CompilerParams(
        use_tc_tiling_on_sc=False, needs_layout_passes=False),
)
def _gather_kernel(idx_hbm, t128_hbm, out_hbm, idx_v, bigs, rows, exts,
                   gsem, osem, isem):
    wid = lax.axis_index("s") * _NC + lax.axis_index("c")
    b0 = wid * _BW
    iota = lax.iota(jnp.int32, 16)
    iota32 = iota * 32

    pltpu.async_copy(idx_hbm.at[:, pl.ds(b0, _BW)], idx_v, isem).wait()

    def hq(g):
        return g // _QPH, g % _QPH

    def fill_big(g, k):
        h, q = hq(g)

        def bj(j, carry):
            v = idx_v[h, pl.ds(q * _C + j * 16, 16)]
            bigs[k][pl.ds(j * 16, 16)] = v >> 2
            return carry

        lax.fori_loop(0, _C // 16, bj, 0)

    def gather_for(g, k):
        return pltpu.make_async_copy(t128_hbm.at[bigs[k]], rows[k], gsem[k])

    def extract(g, k):
        h, q = hq(g)

        def ej(j, carry):
            off = (idx_v[h, pl.ds(q * _C + j * 16, 16)] & 3) * 32
            rowi = j * 16 + iota
            for e0 in range(EMBED_DIM):
                v = plsc.load_gather(rows[k], [rowi, off + e0])
                pos = j * 512 + e0 + iota32
                plsc.store_scatter(exts[k], [pos >> 7, pos & 127], v)
            return carry

        lax.fori_loop(0, _C // 16, ej, 0)

    def out_for(g, k):
        h, q = hq(g)
        r0 = h * (BATCH // 4) + (b0 + q * _C) // 4
        return pltpu.make_async_copy(
            exts[k], out_hbm.at[pl.ds(r0, _C // 4)], osem[k])

    def step(g, k):
        @pl.when(g >= _NBUF)
        def _():
            out_for(g, k).wait()

        fill_big(g, k)
        gather_for(g, k).start()

        km1 = (k + _NBUF - 1) % _NBUF

        @pl.when(g >= 1)
        def _():
            gather_for(g - 1, km1).wait()
            extract(g - 1, km1)
            out_for(g - 1, km1).start()

    def body(i, carry):
        for k in range(_NBUF):
            step(i * _NBUF + k, k)
        return carry

    lax.fori_loop(0, _NCHUNKS // _NBUF, body, 0)

    last = _NCHUNKS - 1
    klast = last % _NBUF
    gather_for(last, klast).wait()
    extract(last, klast)
    out_for(last, klast).start()
    for g in range(_NCHUNKS - _NBUF, _NCHUNKS):
        out_for(g, g % _NBUF).wait()


def kernel(x, table):
    xt = x.T.astype(jnp.int32)
    t128 = table.reshape(VOCAB // 4, 128)
    out = _gather_kernel(xt, t128)
    return out.reshape(HIST, BATCH, EMBED_DIM).transpose(1, 0, 2)


# speedup vs baseline: 1.9615x; 1.9615x over previous
import functools

import jax
import jax.numpy as jnp
from jax import lax
from jax.experimental import pallas as pl
from jax.experimental.pallas import tpu as pltpu
from jax.experimental.pallas import tpu_sc as plsc

VOCAB = 1_000_000
EMBED_DIM = 32
BATCH = 16384
HIST = 50

_NC = 2
_NS = 16
_NW = _NC * _NS

_B = BATCH * HIST
_BW = BATCH // _NW
_NBUF = 4
_NCHUNKS = HIST
_NITER = 48 // _NBUF

_mesh = plsc.VectorSubcoreMesh(core_axis_name="c", subcore_axis_name="s")


@functools.partial(
    pl.kernel,
    mesh=_mesh,
    out_type=jax.ShapeDtypeStruct((_B, EMBED_DIM), jnp.float32),
    scratch_types=[
        pltpu.VMEM((HIST, _BW), jnp.int32),
        [pltpu.VMEM((_BW, EMBED_DIM), jnp.float32) for _ in range(_NBUF)],
        [pltpu.SemaphoreType.DMA for _ in range(_NBUF)],
        [pltpu.SemaphoreType.DMA for _ in range(_NBUF)],
        pltpu.SemaphoreType.DMA,
    ],
    compiler_params=pltpu.CompilerParams(use_tc_tiling_on_sc=False),
)
def _gather_kernel(idx_hbm, table_hbm, out_hbm, idx_v, rows, gsem, osem, isem):
    wid = lax.axis_index("s") * _NC + lax.axis_index("c")
    b0 = wid * _BW

    pltpu.async_copy(idx_hbm.at[:, pl.ds(b0, _BW)], idx_v, isem).wait()

    def gather_for(h, k):
        return pltpu.make_async_copy(
            table_hbm.at[idx_v.at[h]], rows[k], gsem[k])

    def out_for(h, k):
        return pltpu.make_async_copy(
            rows[k], out_hbm.at[pl.ds(h * BATCH + b0, _BW)], osem[k])

    def body(i, carry):
        for k in range(_NBUF):
            h = i * _NBUF + k

            @pl.when(h >= _NBUF)
            def _():
                out_for(h, k).wait()

            gather_for(h, k).start()

            km1 = (k + _NBUF - 1) % _NBUF

            @pl.when(h >= 1)
            def _():
                gather_for(h, km1).wait()
                out_for(h - 1, km1).start()

        return carry

    lax.fori_loop(0, _NITER, body, 0)

    for h in (48, 49):
        k = h % _NBUF
        out_for(h, k).wait()
        gather_for(h, k).start()
        km1 = (k + _NBUF - 1) % _NBUF
        gather_for(h, km1).wait()
        out_for(h - 1, km1).start()
    gather_for(49, 49 % _NBUF).wait()
    out_for(49, 49 % _NBUF).start()
    for h in range(_NCHUNKS - _NBUF, _NCHUNKS):
        out_for(h, h % _NBUF).wait()


def kernel(x, table):
    xt = x.T.astype(jnp.int32)
    tflat = lax.optimization_barrier(table.reshape(-1))
    out = _gather_kernel(xt, tflat.reshape(VOCAB, EMBED_DIM))
    return out.reshape(HIST, BATCH, EMBED_DIM).transpose(1, 0, 2)
